# Initial kernel scaffold; baseline (speedup 1.0000x reference)
import functools
import jax
import jax.numpy as jnp
from jax import lax
from jax.experimental import pallas as pl
from jax.experimental.pallas import tpu as pltpu
from jax.experimental.pallas import tpu_sc as plsc

B = 16384
D = 128
NC = 2
NS = 16
NW = NC * NS
BPW = B // NW
CHUNK = 128
NCHUNK = BPW // CHUNK


def _sc_gather_body(uid_hbm, mid_hbm, gu_t, gm_t, mu_t, mm_t,
                    gu_o, gm_o, mu_o, mm_o,
                    idx_u, idx_m, buf_gu, buf_gm, buf_mu, buf_mm, sem):
    c = lax.axis_index("c")
    s = lax.axis_index("s")
    wid = s * NC + c
    base = wid * BPW
    pltpu.sync_copy(uid_hbm.at[pl.ds(base, BPW)], idx_u)
    pltpu.sync_copy(mid_hbm.at[pl.ds(base, BPW)], idx_m)
    for k in range(NCHUNK):
        iu = idx_u.at[pl.ds(k * CHUNK, CHUNK)]
        im = idx_m.at[pl.ds(k * CHUNK, CHUNK)]
        cp1 = pltpu.async_copy(gu_t.at[iu], buf_gu, sem)
        cp2 = pltpu.async_copy(gm_t.at[im], buf_gm, sem)
        cp3 = pltpu.async_copy(mu_t.at[iu], buf_mu, sem)
        cp4 = pltpu.async_copy(mm_t.at[im], buf_mm, sem)
        cp1.wait()
        cp2.wait()
        cp3.wait()
        cp4.wait()
        rows = pl.ds(base + k * CHUNK, CHUNK)
        pltpu.sync_copy(buf_gu, gu_o.at[rows])
        pltpu.sync_copy(buf_gm, gm_o.at[rows])
        pltpu.sync_copy(buf_mu, mu_o.at[rows])
        pltpu.sync_copy(buf_mm, mm_o.at[rows])


@jax.jit
def _sc_gather(user_ids, movie_ids, gu_t, gm_t, mu_t, mm_t):
    mesh = plsc.VectorSubcoreMesh(core_axis_name="c", subcore_axis_name="s")
    row = jax.ShapeDtypeStruct((B, D), jnp.float32)
    return pl.kernel(
        _sc_gather_body,
        out_type=[row, row, row, row],
        mesh=mesh,
        scratch_types=[
            pltpu.VMEM((BPW,), jnp.int32),
            pltpu.VMEM((BPW,), jnp.int32),
            pltpu.VMEM((CHUNK, D), jnp.float32),
            pltpu.VMEM((CHUNK, D), jnp.float32),
            pltpu.VMEM((CHUNK, D), jnp.float32),
            pltpu.VMEM((CHUNK, D), jnp.float32),
            pltpu.SemaphoreType.DMA,
        ],
    )(user_ids, movie_ids, gu_t, gm_t, mu_t, mm_t)


BT = 2048


def _tc_dense_body(gu, gm, mu, mm, w1u, w1m, b1, w2, b2, wg, wm, bb, out):
    h1 = jnp.maximum(
        jnp.dot(mu[...], w1u[...], preferred_element_type=jnp.float32)
        + jnp.dot(mm[...], w1m[...], preferred_element_type=jnp.float32)
        + b1[...], 0.0)
    h2 = jnp.maximum(
        jnp.dot(h1, w2[...], preferred_element_type=jnp.float32) + b2[...], 0.0)
    g = gu[...] * gm[...]
    out[...] = (jnp.sum(g * wg[...], axis=1)
                + jnp.sum(h2 * wm[...], axis=1) + bb[0])


@jax.jit
def _tc_dense(gu, gm, mu, mm, w1u, w1m, b1, w2, b2, wg, wm, bb):
    row_spec = pl.BlockSpec((BT, D), lambda i: (i, 0))
    full = pl.BlockSpec(lambda i: (0, 0))
    grid = (B // BT,)
    return pl.pallas_call(
        _tc_dense_body,
        grid=grid,
        in_specs=[row_spec, row_spec, row_spec, row_spec,
                  full, full, full, full, full, full, full,
                  pl.BlockSpec(memory_space=pltpu.SMEM)],
        out_specs=pl.BlockSpec((BT,), lambda i: (i,)),
        out_shape=jax.ShapeDtypeStruct((B,), jnp.float32),
    )(gu, gm, mu, mm, w1u, w1m, b1, w2, b2, wg, wm, bb)


def kernel(user_ids, movie_ids, gmf_user_table, gmf_movie_table,
           mlp_user_table, mlp_movie_table, W1, b1, W2, b2, Wout, bout):
    gu, gm, mu, mm = _sc_gather(user_ids, movie_ids, gmf_user_table,
                                gmf_movie_table, mlp_user_table,
                                mlp_movie_table)
    w1u = W1[:, :D].T
    w1m = W1[:, D:].T
    w2 = W2.T
    wg = Wout[:, :D]
    wm = Wout[:, D:]
    return _tc_dense(gu, gm, mu, mm, w1u, w1m, b1.reshape(1, -1),
                     w2, b2.reshape(1, -1), wg, wm, bout)


# speedup vs baseline: 2.6575x; 2.6575x over previous
import functools
import jax
import jax.numpy as jnp
from jax import lax
from jax.experimental import pallas as pl
from jax.experimental.pallas import tpu as pltpu
from jax.experimental.pallas import tpu_sc as plsc

B = 16384
D = 128
NC = 2
NS = 16
NW = NC * NS
BPW = B // NW
CHUNK = 128
NCHUNK = BPW // CHUNK


def _sc_gather_body(uid_hbm, mid_hbm, gu_t, gm_t, mu_t, mm_t,
                    gu_o, gm_o, mu_o, mm_o,
                    idx_u, idx_m, buf_gu, buf_gm, buf_mu, buf_mm, sem):
    c = lax.axis_index("c")
    s = lax.axis_index("s")
    wid = s * NC + c
    base = wid * BPW
    pltpu.sync_copy(uid_hbm.at[pl.ds(base, BPW)], idx_u)
    pltpu.sync_copy(mid_hbm.at[pl.ds(base, BPW)], idx_m)
    for k in range(NCHUNK):
        iu = idx_u.at[pl.ds(k * CHUNK, CHUNK)]
        im = idx_m.at[pl.ds(k * CHUNK, CHUNK)]
        cp1 = pltpu.async_copy(gu_t.at[iu], buf_gu, sem)
        cp2 = pltpu.async_copy(gm_t.at[im], buf_gm, sem)
        cp3 = pltpu.async_copy(mu_t.at[iu], buf_mu, sem)
        cp4 = pltpu.async_copy(mm_t.at[im], buf_mm, sem)
        cp1.wait()
        cp2.wait()
        cp3.wait()
        cp4.wait()
        rows = pl.ds(base + k * CHUNK, CHUNK)
        pltpu.sync_copy(buf_gu, gu_o.at[rows])
        pltpu.sync_copy(buf_gm, gm_o.at[rows])
        pltpu.sync_copy(buf_mu, mu_o.at[rows])
        pltpu.sync_copy(buf_mm, mm_o.at[rows])


@jax.jit
def _sc_gather(user_ids, movie_ids, gu_t, gm_t, mu_t, mm_t):
    mesh = plsc.VectorSubcoreMesh(core_axis_name="c", subcore_axis_name="s",
                                  num_cores=NC, num_subcores=NS)
    row = jax.ShapeDtypeStruct((B, D), jnp.float32)
    return pl.kernel(
        _sc_gather_body,
        out_type=[row, row, row, row],
        mesh=mesh,
        scratch_types=[
            pltpu.VMEM((BPW,), jnp.int32),
            pltpu.VMEM((BPW,), jnp.int32),
            pltpu.VMEM((CHUNK, D), jnp.float32),
            pltpu.VMEM((CHUNK, D), jnp.float32),
            pltpu.VMEM((CHUNK, D), jnp.float32),
            pltpu.VMEM((CHUNK, D), jnp.float32),
            pltpu.SemaphoreType.DMA,
        ],
    )(user_ids, movie_ids, gu_t, gm_t, mu_t, mm_t)


BT = 2048


def _tc_dense_body(gu, gm, mu, mm, w1u, w1m, b1, w2, b2, wg, wm, bb, out):
    h1 = jnp.maximum(
        jnp.dot(mu[...], w1u[...], preferred_element_type=jnp.float32)
        + jnp.dot(mm[...], w1m[...], preferred_element_type=jnp.float32)
        + b1[...], 0.0)
    h2 = jnp.maximum(
        jnp.dot(h1, w2[...], preferred_element_type=jnp.float32) + b2[...], 0.0)
    g = gu[...] * gm[...]
    out[...] = (jnp.sum(g * wg[...], axis=1)
                + jnp.sum(h2 * wm[...], axis=1) + bb[0])


@jax.jit
def _tc_dense(gu, gm, mu, mm, w1u, w1m, b1, w2, b2, wg, wm, bb):
    row_spec = pl.BlockSpec((BT, D), lambda i: (i, 0))

    def full(shape):
        return pl.BlockSpec(shape, lambda i: (0, 0))

    grid = (B // BT,)
    return pl.pallas_call(
        _tc_dense_body,
        grid=grid,
        in_specs=[row_spec, row_spec, row_spec, row_spec,
                  full((D, 64)), full((D, 64)), full((1, 64)),
                  full((64, D)), full((1, D)), full((1, D)), full((1, D)),
                  pl.BlockSpec(memory_space=pltpu.SMEM)],
        out_specs=pl.BlockSpec((BT,), lambda i: (i,)),
        out_shape=jax.ShapeDtypeStruct((B,), jnp.float32),
    )(gu, gm, mu, mm, w1u, w1m, b1, w2, b2, wg, wm, bb)


def kernel(user_ids, movie_ids, gmf_user_table, gmf_movie_table,
           mlp_user_table, mlp_movie_table, W1, b1, W2, b2, Wout, bout):
    gu, gm, mu, mm = _sc_gather(user_ids, movie_ids, gmf_user_table,
                                gmf_movie_table, mlp_user_table,
                                mlp_movie_table)
    w1u = W1[:, :D].T
    w1m = W1[:, D:].T
    w2 = W2.T
    wg = Wout[:, :D]
    wm = Wout[:, D:]
    return _tc_dense(gu, gm, mu, mm, w1u, w1m, b1.reshape(1, -1),
                     w2, b2.reshape(1, -1), wg, wm, bout)
